# baseline (speedup 1.0000x reference)
import functools

import numpy as np
import jax
import jax.numpy as jnp
from jax import lax
from jax.experimental import pallas as pl
from jax.experimental.pallas import tpu as pltpu
from jax.experimental.pallas import tpu_sc as plsc

_VOCAB = 1000000
_D = 64
_B = 1024
_S = 200
_T = _B * _S
_NELEM = _T * _D
_SCALE = np.float32(5.0 / np.sqrt(_S * _D))

_LANES = 128
_NROWS = _NELEM // _LANES

_BB = 32
_HTOK = _BB * _S // 2
_BROWS = _HTOK


def _threefry_eps(p):
    ks0 = jnp.uint32(0)
    ks1 = jnp.uint32(1)
    ks2 = jnp.uint32(0x1BD11BDB)
    x0 = jnp.full_like(p, ks0)
    x1 = p + ks1
    rot0 = (13, 15, 26, 6)
    rot1 = (17, 29, 16, 24)
    schedule = (
        (rot0, ks1, ks2, 1),
        (rot1, ks2, ks0, 2),
        (rot0, ks0, ks1, 3),
        (rot1, ks1, ks2, 4),
        (rot0, ks2, ks0, 5),
    )
    for rots, ka, kb, c in schedule:
        for r in rots:
            x0 = x0 + x1
            x1 = (x1 << jnp.uint32(r)) | (x1 >> jnp.uint32(32 - r))
            x1 = x0 ^ x1
        x0 = x0 + ka
        x1 = x1 + kb + jnp.uint32(c)
    bits = x0 ^ x1
    fbits = (bits >> jnp.uint32(9)) | jnp.uint32(0x3F800000)
    return lax.bitcast_convert_type(fbits, jnp.float32) - jnp.float32(1.0)


def _eps_body(o_ref):
    b = pl.program_id(0)
    base = b.astype(jnp.uint32) * jnp.uint32(2 * _HTOK * _D)
    i = lax.broadcasted_iota(jnp.uint32, (_BROWS, _LANES), 0)
    j = lax.broadcasted_iota(jnp.uint32, (_BROWS, _LANES), 1)
    p = base + i * jnp.uint32(_D) + j + jnp.where(
        j < _D, jnp.uint32(0), jnp.uint32(_HTOK * _D - _D))
    o_ref[...] = _SCALE * _threefry_eps(p)


def _eps2d(interpret=False):
    return pl.pallas_call(
        _eps_body,
        grid=(_B // _BB,),
        out_specs=pl.BlockSpec((_BROWS, _LANES), lambda b: (b, 0)),
        out_shape=jax.ShapeDtypeStruct((_NROWS, _LANES), jnp.float32),
        interpret=interpret,
    )()


def _add_body(x_ref, e_ref, o_ref):
    y = x_ref[...] + e_ref[...]
    hb = _BB // 2
    o_ref[pl.ds(0, hb), :, :] = y[:, :_D].reshape(hb, _S, _D)
    o_ref[pl.ds(hb, hb), :, :] = y[:, _D:].reshape(hb, _S, _D)


def _add3d(g2d, eps, interpret=False):
    return pl.pallas_call(
        _add_body,
        grid=(_B // _BB,),
        in_specs=[pl.BlockSpec((_BROWS, _LANES), lambda b: (b, 0)),
                  pl.BlockSpec((_BROWS, _LANES), lambda b: (b, 0))],
        out_specs=pl.BlockSpec((_BB, _S, _D), lambda b: (b, 0, 0)),
        out_shape=jax.ShapeDtypeStruct((_B, _S, _D), jnp.float32),
        interpret=interpret,
    )(g2d, eps)


_NW = 32
_NVB = _VOCAB // _LANES
_K1ITER = (_NVB + _NW - 1) // _NW
_PROWS = _VOCAB // 2


def _sc_transpose_body(tabt_hbm, tail_hbm, out_hbm, tin, tout, isem, osem):
    w = lax.axis_index("s") * 2 + lax.axis_index("c")

    @pl.when(w == 0)
    def _():
        pltpu.sync_copy(tail_hbm, out_hbm.at[pl.ds(_PROWS - 32, 32)])

    iota16 = lax.iota(jnp.int32, 16)

    def fetch(i):
        blk = i * _NW + w
        s = i % 2

        @pl.when(blk < _NVB)
        def _():
            col = pl.multiple_of(blk * _LANES, _LANES)
            pltpu.async_copy(tabt_hbm.at[:, pl.ds(col, _LANES)],
                             tin.at[s], isem.at[s])

    fetch(0)

    def body(i, carry):
        blk = i * _NW + w
        s = i % 2

        @pl.when(blk < _NVB)
        def _():
            pltpu.make_async_copy(tabt_hbm.at[:, pl.ds(0, _LANES)],
                                  tin.at[s], isem.at[s]).wait()
        fetch(i + 1)

        @pl.when(blk < _NVB)
        def _():
            @pl.when(i >= 2)
            def _():
                pltpu.make_async_copy(tout.at[s],
                                      out_hbm.at[pl.ds(0, _D)],
                                      osem.at[s]).wait()

            def prow(pr, c2):
                for k in range(8):
                    v = 2 * pr + (1 if k >= 4 else 0)
                    vec = plsc.load_gather(
                        tin.at[s],
                        [iota16 + 16 * (k % 4), jnp.full((16,), 0, jnp.int32) + v])
                    tout[s, pr, pl.ds(16 * k, 16)] = vec
                return c2
            lax.fori_loop(0, _D, prow, 0)
            orow = pl.multiple_of(blk * _D, 8)
            pltpu.async_copy(tout.at[s], out_hbm.at[pl.ds(orow, _D)],
                             osem.at[s])
        return carry

    lax.fori_loop(0, _K1ITER, body, 0)
    for i_last in (_K1ITER - 2, _K1ITER - 1):
        s = i_last % 2

        @pl.when(i_last * _NW + w < _NVB)
        def _():
            pltpu.make_async_copy(tout.at[s],
                                  out_hbm.at[pl.ds(0, _D)], osem.at[s]).wait()


def _sc_transpose(tabt, tail2d):
    mesh = plsc.VectorSubcoreMesh(core_axis_name="c", subcore_axis_name="s")
    scratch = [
        pltpu.VMEM((2, _D, _LANES), jnp.float32),
        pltpu.VMEM((2, _D, _LANES), jnp.float32),
        pltpu.SemaphoreType.DMA((2,)),
        pltpu.SemaphoreType.DMA((2,)),
    ]
    k = pl.kernel(
        _sc_transpose_body,
        out_type=jax.ShapeDtypeStruct((_PROWS, _LANES), jnp.float32),
        mesh=mesh,
        scratch_types=scratch,
        compiler_params=pltpu.CompilerParams(needs_layout_passes=False),
    )
    return k(tabt, tail2d)


_TPW = _T // _NW
_CHUNK = 128
_HCH = _TPW // 2 // _CHUNK
_IDROWS = 56


def _sc_gather_body(idx_hbm, table_hbm, out_hbm, idx_v, gbuf, obuf,
                    gsem, osem):
    w = lax.axis_index("s") * 2 + lax.axis_index("c")
    row_base = w * (_TPW // 2)
    pltpu.sync_copy(idx_hbm.at[pl.ds(w * _IDROWS, _IDROWS)], idx_v)

    iota16 = lax.iota(jnp.int32, 16)

    def start_gather(q):
        s = q % 2
        for h in range(2):
            for m in range(_CHUNK // 16):
                idxvec = idx_v[q + _HCH * h, pl.ds(16 * m, 16)] >> 1
                pltpu.async_copy(table_hbm.at[idxvec],
                                 gbuf.at[s, h, pl.ds(16 * m, 16)],
                                 gsem.at[s])

    def wait_gather(q):
        s = q % 2
        for h in range(2):
            for m in range(_CHUNK // 16):
                pltpu.make_async_copy(table_hbm.at[iota16],
                                      gbuf.at[s, h, pl.ds(16 * m, 16)],
                                      gsem.at[s]).wait()

    start_gather(0)
    start_gather(1)
    odesc = [None] * _HCH
    for q in range(_HCH):
        s = q % 2
        wait_gather(q)
        if q >= 2:
            odesc[q - 2].wait()

        def tok(t, c2):
            rowv = jnp.full((16,), 0, jnp.int32) + t
            for h in range(2):
                vvec = plsc.load_gather(
                    idx_v, [jnp.full((16,), q + _HCH * h, jnp.int32), rowv])
                offv = (vvec & 1) << 6
                for k in range(4):
                    vec = plsc.load_gather(
                        gbuf.at[s, h], [rowv, offv + (16 * k) + iota16])
                    obuf[s, t, pl.ds(_D * h + 16 * k, 16)] = vec
            return c2
        lax.fori_loop(0, _CHUNK, tok, 0)

        orow = row_base + q * _CHUNK
        odesc[q] = pltpu.async_copy(
            obuf.at[s], out_hbm.at[pl.ds(orow, _CHUNK)], osem.at[s])
        if q + 2 < _HCH:
            start_gather(q + 2)
    odesc[_HCH - 2].wait()
    odesc[_HCH - 1].wait()


def _sc_gather(ids_padded, table_pairs):
    mesh = plsc.VectorSubcoreMesh(core_axis_name="c", subcore_axis_name="s")
    scratch = [
        pltpu.VMEM((_IDROWS, _LANES), jnp.int32),
        pltpu.VMEM((2, 2, _CHUNK, _LANES), jnp.float32),
        pltpu.VMEM((2, _CHUNK, _LANES), jnp.float32),
        pltpu.SemaphoreType.DMA((2,)),
        pltpu.SemaphoreType.DMA((2,)),
    ]
    k = pl.kernel(
        _sc_gather_body,
        out_type=jax.ShapeDtypeStruct((_NROWS, _LANES), jnp.float32),
        mesh=mesh,
        scratch_types=scratch,
        compiler_params=pltpu.CompilerParams(needs_layout_passes=False),
    )
    return k(ids_padded, table_pairs)


def kernel(input_ids, table):
    tabt = table.T
    tail2d = table[_VOCAB - 64:, :].reshape(32, _LANES)
    tabp = _sc_transpose(tabt, tail2d)
    ids2 = jnp.concatenate(
        [input_ids.reshape(_NW, _TPW),
         jnp.zeros((_NW, _IDROWS * _LANES - _TPW), jnp.int32)],
        axis=1).reshape(_NW * _IDROWS, _LANES)
    g2d = _sc_gather(ids2, tabp)
    eps = _eps2d()
    return _add3d(g2d, eps)


# speedup vs baseline: 1.9461x; 1.9461x over previous
import functools

import numpy as np
import jax
import jax.numpy as jnp
from jax import lax
from jax.experimental import pallas as pl
from jax.experimental.pallas import tpu as pltpu
from jax.experimental.pallas import tpu_sc as plsc

_VOCAB = 1000000
_D = 64
_B = 1024
_S = 200
_T = _B * _S
_NELEM = _T * _D
_SCALE = np.float32(5.0 / np.sqrt(_S * _D))

_LANES = 128
_NROWS = _NELEM // _LANES

_BB = 32
_HTOK = _BB * _S // 2
_BROWS = _HTOK


def _threefry_eps(p):
    ks0 = jnp.uint32(0)
    ks1 = jnp.uint32(1)
    ks2 = jnp.uint32(0x1BD11BDB)
    x0 = jnp.full_like(p, ks0)
    x1 = p + ks1
    rot0 = (13, 15, 26, 6)
    rot1 = (17, 29, 16, 24)
    schedule = (
        (rot0, ks1, ks2, 1),
        (rot1, ks2, ks0, 2),
        (rot0, ks0, ks1, 3),
        (rot1, ks1, ks2, 4),
        (rot0, ks2, ks0, 5),
    )
    for rots, ka, kb, c in schedule:
        for r in rots:
            x0 = x0 + x1
            x1 = (x1 << jnp.uint32(r)) | (x1 >> jnp.uint32(32 - r))
            x1 = x0 ^ x1
        x0 = x0 + ka
        x1 = x1 + kb + jnp.uint32(c)
    bits = x0 ^ x1
    fbits = (bits >> jnp.uint32(9)) | jnp.uint32(0x3F800000)
    return lax.bitcast_convert_type(fbits, jnp.float32) - jnp.float32(1.0)


def _eps_body(o_ref):
    b = pl.program_id(0)
    base = b.astype(jnp.uint32) * jnp.uint32(2 * _HTOK * _D)
    i = lax.broadcasted_iota(jnp.uint32, (_BROWS, _LANES), 0)
    j = lax.broadcasted_iota(jnp.uint32, (_BROWS, _LANES), 1)
    p = base + i * jnp.uint32(_D) + j + jnp.where(
        j < _D, jnp.uint32(0), jnp.uint32(_HTOK * _D - _D))
    o_ref[...] = _SCALE * _threefry_eps(p)


def _eps2d(interpret=False):
    return pl.pallas_call(
        _eps_body,
        grid=(_B // _BB,),
        out_specs=pl.BlockSpec((_BROWS, _LANES), lambda b: (b, 0)),
        out_shape=jax.ShapeDtypeStruct((_NROWS, _LANES), jnp.float32),
        interpret=interpret,
    )()


def _add_body(x_ref, e_ref, o_ref):
    y = x_ref[...] + e_ref[...]
    hb = _BB // 2
    o_ref[pl.ds(0, hb), :, :] = y[:, :_D].reshape(hb, _S, _D)
    o_ref[pl.ds(hb, hb), :, :] = y[:, _D:].reshape(hb, _S, _D)


def _add3d(g2d, eps, interpret=False):
    return pl.pallas_call(
        _add_body,
        grid=(_B // _BB,),
        in_specs=[pl.BlockSpec((_BROWS, _LANES), lambda b: (b, 0)),
                  pl.BlockSpec((_BROWS, _LANES), lambda b: (b, 0))],
        out_specs=pl.BlockSpec((_BB, _S, _D), lambda b: (b, 0, 0)),
        out_shape=jax.ShapeDtypeStruct((_B, _S, _D), jnp.float32),
        interpret=interpret,
    )(g2d, eps)


_NW = 32
_TPW = _T // _NW
_IDR = _TPW // _S
_HIDR = _IDR // 2
_CSZ = (96, 104)
_NCH = 2 * _IDR
_NBUF = 4


def _sc_gather_body(idx_hbm, table_hbm, out_hbm, idx_v, bufs, *sems):
    gsems = sems[:_NBUF]
    osems = sems[_NBUF:]
    w = lax.axis_index("s") * 2 + lax.axis_index("c")
    row_base = w * (_TPW // 2)
    pltpu.sync_copy(idx_hbm.at[pl.ds(w * _IDR, _IDR)], idx_v)

    def start_gather(j):
        b = j % _NBUF
        r, par = divmod(j, 2)
        sz = _CSZ[par]
        idx = idx_v.at[r, pl.ds(par * _CSZ[0], sz)]
        return pltpu.async_copy(table_hbm.at[idx],
                                bufs.at[b, pl.ds(0, sz)], gsems[b])

    def start_out(j):
        b = j % _NBUF
        r, par = divmod(j, 2)
        sz = _CSZ[par]
        half, rr = divmod(r, _HIDR)
        row = row_base + rr * _S + par * _CSZ[0]
        dst = out_hbm.at[pl.ds(row, sz), pl.ds(half * _D, _D)]
        return pltpu.async_copy(bufs.at[b, pl.ds(0, sz)], dst, osems[b])

    gdesc = [None] * _NCH
    odesc = [None] * _NCH
    for j in range(min(2, _NCH)):
        gdesc[j] = start_gather(j)
    for j in range(_NCH):
        gdesc[j].wait()
        odesc[j] = start_out(j)
        nj = j + 2
        if nj < _NCH:
            if nj - _NBUF >= 0:
                odesc[nj - _NBUF].wait()
            gdesc[nj] = start_gather(nj)
    for j in range(_NCH - _NBUF, _NCH):
        odesc[j].wait()


def _sc_gather(input_ids, table):
    mesh = plsc.VectorSubcoreMesh(core_axis_name="c", subcore_axis_name="s")
    scratch = [
        pltpu.VMEM((_IDR, _S), jnp.int32),
        pltpu.VMEM((_NBUF, _CSZ[1], _D), jnp.float32),
    ] + [pltpu.SemaphoreType.DMA] * (2 * _NBUF)
    k = pl.kernel(
        _sc_gather_body,
        out_type=jax.ShapeDtypeStruct((_NROWS, _LANES), jnp.float32),
        mesh=mesh,
        scratch_types=scratch,
        compiler_params=pltpu.CompilerParams(use_tc_tiling_on_sc=False),
    )
    return k(input_ids, table)


def kernel(input_ids, table):
    g2d = _sc_gather(input_ids, table)
    eps = _eps2d()
    return _add3d(g2d, eps)


# speedup vs baseline: 1.9762x; 1.0154x over previous
import functools

import numpy as np
import jax
import jax.numpy as jnp
from jax import lax
from jax.experimental import pallas as pl
from jax.experimental.pallas import tpu as pltpu
from jax.experimental.pallas import tpu_sc as plsc

_VOCAB = 1000000
_D = 64
_B = 1024
_S = 200
_T = _B * _S
_NELEM = _T * _D
_SCALE = np.float32(5.0 / np.sqrt(_S * _D))

_LANES = 128
_NROWS = _NELEM // _LANES
_BLK = 512
_GRID = _NROWS // _BLK


def _threefry_eps(p):
    ks0 = jnp.uint32(0)
    ks1 = jnp.uint32(1)
    ks2 = jnp.uint32(0x1BD11BDB)
    x0 = jnp.full_like(p, ks0)
    x1 = p + ks1
    rot0 = (13, 15, 26, 6)
    rot1 = (17, 29, 16, 24)
    schedule = (
        (rot0, ks1, ks2, 1),
        (rot1, ks2, ks0, 2),
        (rot0, ks0, ks1, 3),
        (rot1, ks1, ks2, 4),
        (rot0, ks2, ks0, 5),
    )
    for rots, ka, kb, c in schedule:
        for r in rots:
            x0 = x0 + x1
            x1 = (x1 << jnp.uint32(r)) | (x1 >> jnp.uint32(32 - r))
            x1 = x0 ^ x1
        x0 = x0 + ka
        x1 = x1 + kb + jnp.uint32(c)
    bits = x0 ^ x1
    fbits = (bits >> jnp.uint32(9)) | jnp.uint32(0x3F800000)
    return lax.bitcast_convert_type(fbits, jnp.float32) - jnp.float32(1.0)


_BB = 32
_HTOK = _BB * _S // 2
_BROWS = _HTOK


def _noise_add_body(x_ref, o_ref):
    b = pl.program_id(0)
    base = b.astype(jnp.uint32) * jnp.uint32(2 * _HTOK * _D)
    i = lax.broadcasted_iota(jnp.uint32, (_BROWS, _LANES), 0)
    j = lax.broadcasted_iota(jnp.uint32, (_BROWS, _LANES), 1)
    p = base + i * jnp.uint32(_D) + j + jnp.where(
        j < _D, jnp.uint32(0), jnp.uint32(_HTOK * _D - _D))
    y = x_ref[...] + _SCALE * _threefry_eps(p)
    hb = _BB // 2
    o_ref[pl.ds(0, hb), :, :] = y[:, :_D].reshape(hb, _S, _D)
    o_ref[pl.ds(hb, hb), :, :] = y[:, _D:].reshape(hb, _S, _D)


def _noise_add(xs2d, interpret=False):
    return pl.pallas_call(
        _noise_add_body,
        grid=(_B // _BB,),
        in_specs=[pl.BlockSpec((_BROWS, _LANES), lambda b: (b, 0))],
        out_specs=pl.BlockSpec((_BB, _S, _D), lambda b: (b, 0, 0)),
        out_shape=jax.ShapeDtypeStruct((_B, _S, _D), jnp.float32),
        interpret=interpret,
    )(xs2d)


_NW = 32
_TPW = _T // _NW
_IDR = _TPW // _S
_HIDR = _IDR // 2
_CSZ = (96, 104)
_NCH = 2 * _IDR
_NBUF = 4


def _sc_gather_body(idx_hbm, table_hbm, out_hbm, idx_v, bufs, *sems):
    gsems = sems[:_NBUF]
    osems = sems[_NBUF:]
    w = lax.axis_index("s") * 2 + lax.axis_index("c")
    row_base = w * (_TPW // 2)
    pltpu.sync_copy(idx_hbm.at[pl.ds(w * _IDR, _IDR)], idx_v)

    def start_gather(j):
        b = j % _NBUF
        r, par = divmod(j, 2)
        sz = _CSZ[par]
        idx = idx_v.at[r, pl.ds(par * _CSZ[0], sz)]
        return pltpu.async_copy(table_hbm.at[idx],
                                bufs.at[b, pl.ds(0, sz)], gsems[b])

    def start_out(j):
        b = j % _NBUF
        r, par = divmod(j, 2)
        sz = _CSZ[par]
        half, rr = divmod(r, _HIDR)
        row = row_base + rr * _S + par * _CSZ[0]
        dst = out_hbm.at[pl.ds(row, sz), pl.ds(half * _D, _D)]
        return pltpu.async_copy(bufs.at[b, pl.ds(0, sz)], dst, osems[b])

    gdesc = [None] * _NCH
    odesc = [None] * _NCH
    for j in range(min(2, _NCH)):
        gdesc[j] = start_gather(j)
    for j in range(_NCH):
        gdesc[j].wait()
        odesc[j] = start_out(j)
        nj = j + 2
        if nj < _NCH:
            if nj - _NBUF >= 0:
                odesc[nj - _NBUF].wait()
            gdesc[nj] = start_gather(nj)
    for j in range(_NCH - _NBUF, _NCH):
        odesc[j].wait()


def _sc_gather(input_ids, table):
    mesh = plsc.VectorSubcoreMesh(core_axis_name="c", subcore_axis_name="s")
    scratch = [
        pltpu.VMEM((_IDR, _S), jnp.int32),
        pltpu.VMEM((_NBUF, _CSZ[1], _D), jnp.float32),
    ] + [pltpu.SemaphoreType.DMA] * (2 * _NBUF)
    k = pl.kernel(
        _sc_gather_body,
        out_type=jax.ShapeDtypeStruct((_NROWS, _LANES), jnp.float32),
        mesh=mesh,
        scratch_types=scratch,
        compiler_params=pltpu.CompilerParams(use_tc_tiling_on_sc=False),
    )
    return k(input_ids, table)


def kernel(input_ids, table):
    g2d = _sc_gather(input_ids, table)
    return _noise_add(g2d)
